# baseline (speedup 1.0000x reference)
import jax
import jax.numpy as jnp
from jax import lax
from jax.experimental import pallas as pl
from jax.experimental.pallas import tpu as pltpu

_N = 10000
_E = 160000
_IN = 256
_OUT = 256
_ED = 16
_H = 512

_EDGE_BLK = 2000
_NODE_BLK = 1000


def _edge_enc_body(ea_ref, we_ref, be_ref, out_ref):
    e = jnp.dot(ea_ref[...], we_ref[...], preferred_element_type=jnp.float32)
    out_ref[...] = jnp.maximum(e + be_ref[...], 0.0)


def _edge_encoder(edge_attr, W_e, b_e):
    grid = (_E // _EDGE_BLK,)
    return pl.pallas_call(
        _edge_enc_body,
        grid=grid,
        in_specs=[
            pl.BlockSpec((_EDGE_BLK, _ED), lambda i: (i, 0)),
            pl.BlockSpec((_ED, _IN), lambda i: (0, 0)),
            pl.BlockSpec((1, _IN), lambda i: (0, 0)),
        ],
        out_specs=pl.BlockSpec((_EDGE_BLK, _IN), lambda i: (i, 0)),
        out_shape=jax.ShapeDtypeStruct((_E, _IN), jnp.float32),
    )(edge_attr, W_e, b_e.reshape(1, _IN))


def _mlp_body(x_ref, agg_ref, deg_ref, w0_ref, b0_ref, w1_ref, b1_ref,
              w2_ref, b2_ref, out_ref):
    deg = jnp.maximum(deg_ref[...], 1.0)
    agg = agg_ref[...] / deg
    h = jnp.concatenate([x_ref[...], agg], axis=-1)
    h = jnp.maximum(jnp.dot(h, w0_ref[...], preferred_element_type=jnp.float32)
                    + b0_ref[...], 0.0)
    h = jnp.maximum(jnp.dot(h, w1_ref[...], preferred_element_type=jnp.float32)
                    + b1_ref[...], 0.0)
    out_ref[...] = jnp.dot(h, w2_ref[...], preferred_element_type=jnp.float32) \
        + b2_ref[...]


def _mlp(x, agg, deg, W0, b0, W1, b1, W2, b2):
    grid = (_N // _NODE_BLK,)
    return pl.pallas_call(
        _mlp_body,
        grid=grid,
        in_specs=[
            pl.BlockSpec((_NODE_BLK, _IN), lambda i: (i, 0)),
            pl.BlockSpec((_NODE_BLK, _IN), lambda i: (i, 0)),
            pl.BlockSpec((_NODE_BLK, 1), lambda i: (i, 0)),
            pl.BlockSpec((2 * _IN, _H), lambda i: (0, 0)),
            pl.BlockSpec((1, _H), lambda i: (0, 0)),
            pl.BlockSpec((_H, _H), lambda i: (0, 0)),
            pl.BlockSpec((1, _H), lambda i: (0, 0)),
            pl.BlockSpec((_H, _OUT), lambda i: (0, 0)),
            pl.BlockSpec((1, _OUT), lambda i: (0, 0)),
        ],
        out_specs=pl.BlockSpec((_NODE_BLK, _OUT), lambda i: (i, 0)),
        out_shape=jax.ShapeDtypeStruct((_N, _OUT), jnp.float32),
    )(x, agg, deg, W0, b0.reshape(1, _H), W1, b1.reshape(1, _H),
      W2, b2.reshape(1, _OUT))


def kernel(x, edge_index, edge_attr, W_e, b_e, W0, b0, W1, b1, W2, b2):
    src = edge_index[0]
    dst = edge_index[1]
    e = _edge_encoder(edge_attr, W_e, b_e)
    m = jnp.maximum(x[src] + e, 0.0)
    agg = jax.ops.segment_sum(m, dst, num_segments=_N)
    deg = jax.ops.segment_sum(jnp.ones((_E, 1), jnp.float32), dst,
                              num_segments=_N)
    return _mlp(x, agg, deg, W0, b0, W1, b1, W2, b2)


# speedup vs baseline: 2.3166x; 2.3166x over previous
import functools

import jax
import jax.numpy as jnp
from jax import lax
from jax.experimental import pallas as pl
from jax.experimental.pallas import tpu as pltpu
from jax.experimental.pallas import tpu_sc as plsc

_N = 10000
_E = 160000
_IN = 256
_OUT = 256
_ED = 16
_H = 512

_HF = 128
_K = 80
_EPT = _E // 16
_NPAD = 10240
_NPT = _NPAD // 16

_EDGE_BLK = 2000
_NODE_BLK = 1000


def _edge_enc_body(ea_ref, we_ref, be_ref, lo_ref, hi_ref):
    e = jnp.dot(ea_ref[...], we_ref[...], preferred_element_type=jnp.float32)
    e = jnp.maximum(e + be_ref[...], 0.0)
    lo_ref[...] = e[:, :_HF]
    hi_ref[...] = e[:, _HF:]


def _edge_encoder(edge_attr, W_e, b_e):
    grid = (_E // _EDGE_BLK,)
    return pl.pallas_call(
        _edge_enc_body,
        grid=grid,
        in_specs=[
            pl.BlockSpec((_EDGE_BLK, _ED), lambda i: (i, 0)),
            pl.BlockSpec((_ED, _IN), lambda i: (0, 0)),
            pl.BlockSpec((1, _IN), lambda i: (0, 0)),
        ],
        out_specs=[
            pl.BlockSpec((_EDGE_BLK, _HF), lambda i: (i, 0)),
            pl.BlockSpec((_EDGE_BLK, _HF), lambda i: (i, 0)),
        ],
        out_shape=[
            jax.ShapeDtypeStruct((_E, _HF), jnp.float32),
            jax.ShapeDtypeStruct((_E, _HF), jnp.float32),
        ],
    )(edge_attr, W_e, b_e.reshape(1, _IN))


def _sc_edges(x_hbm, e_hbm, src_hbm, dst_hbm,
              srcb, dstb, xg, eb, msg, degl, acc, sem, s, do_deg):
    ebase = s * _EPT
    lanes = lax.iota(jnp.int32, 16)
    onev = jnp.ones((16,), jnp.float32)

    def chunk(j, carry):
        b = ebase + j * _K
        pltpu.sync_copy(src_hbm.at[pl.ds(b, _K)], srcb)
        pltpu.sync_copy(dst_hbm.at[pl.ds(b, _K)], dstb)
        pltpu.async_copy(x_hbm.at[srcb], xg, sem).wait()
        pltpu.sync_copy(e_hbm.at[pl.ds(b, _K)], eb)

        def row(k, c2):
            for q in range(_HF // 16):
                xv = xg[k, pl.ds(q * 16, 16)]
                ev = eb[k, pl.ds(q * 16, 16)]
                msg[k, pl.ds(q * 16, 16)] = jnp.maximum(xv + ev, 0.0)
            return c2
        lax.fori_loop(0, _K, row, 0, unroll=False)

        pltpu.sync_copy(msg, acc.at[dstb], add=True)

        if do_deg:
            for v in range(_K // 16):
                dv = dstb[pl.ds(v * 16, 16)]
                for ln in range(16):
                    plsc.addupdate_scatter(degl, [dv], onev,
                                           mask=lanes == ln)
        return carry

    lax.fori_loop(0, _EPT // _K, chunk, 0, unroll=False)


def _dump_acc(acc, out_hbm, msg, s):
    nbase = s * _NPT
    def dump(i, carry):
        r = nbase + i * _K
        pltpu.sync_copy(acc.at[pl.ds(r, _K)], msg)
        pltpu.sync_copy(msg, out_hbm.at[pl.ds(r, _K)])
        return carry
    lax.fori_loop(0, _NPT // _K, dump, 0, unroll=False)


def _sc_body(x0, x1, e0, e1, src_hbm, dst_hbm, out0, out1, outh,
             srcb, dstb, xg, eb, msg, degl, acc, sem):
    c = lax.axis_index("c")
    s = lax.axis_index("s")
    zv = jnp.zeros((16,), jnp.float32)

    def zrow(k, carry):
        for q in range(_HF // 16):
            msg[k, pl.ds(q * 16, 16)] = zv
        return carry
    lax.fori_loop(0, _K, zrow, 0, unroll=False)

    nbase = s * _NPT
    def zinit(i, carry):
        pltpu.sync_copy(msg, acc.at[pl.ds(nbase + i * _K, _K)])
        return carry
    lax.fori_loop(0, _NPT // _K, zinit, 0, unroll=False)

    @pl.when(c == 0)
    def _():
        def zdeg(i, carry):
            degl[pl.ds(i * 16, 16)] = zv
            return carry
        lax.fori_loop(0, _NPAD // 16, zdeg, 0, unroll=False)

    plsc.subcore_barrier()

    @pl.when(c == 0)
    def _():
        _sc_edges(x0, e0, src_hbm, dst_hbm,
                  srcb, dstb, xg, eb, msg, degl, acc, sem, s, True)

    @pl.when(c == 1)
    def _():
        _sc_edges(x1, e1, src_hbm, dst_hbm,
                  srcb, dstb, xg, eb, msg, degl, acc, sem, s, False)

    plsc.subcore_barrier()

    @pl.when(c == 0)
    def _():
        _dump_acc(acc, out0, msg, s)
        pltpu.sync_copy(degl, outh.at[pl.ds(s * _NPAD, _NPAD)])

    @pl.when(c == 1)
    def _():
        _dump_acc(acc, out1, msg, s)


def _sc_aggregate(x0, x1, e0, e1, src, dst):
    mesh = plsc.VectorSubcoreMesh(core_axis_name="c", subcore_axis_name="s")
    fn = functools.partial(
        pl.kernel,
        out_type=[
            jax.ShapeDtypeStruct((_NPAD, _HF), jnp.float32),
            jax.ShapeDtypeStruct((_NPAD, _HF), jnp.float32),
            jax.ShapeDtypeStruct((16 * _NPAD,), jnp.float32),
        ],
        mesh=mesh,
        compiler_params=pltpu.CompilerParams(needs_layout_passes=False),
        scratch_types=[
            pltpu.VMEM((_K,), jnp.int32),
            pltpu.VMEM((_K,), jnp.int32),
            pltpu.VMEM((_K, _HF), jnp.float32),
            pltpu.VMEM((_K, _HF), jnp.float32),
            pltpu.VMEM((_K, _HF), jnp.float32),
            pltpu.VMEM((_NPAD,), jnp.float32),
            pltpu.VMEM_SHARED((_NPAD, _HF), jnp.float32),
            pltpu.SemaphoreType.DMA,
        ],
    )(_sc_body)
    return fn(x0, x1, e0, e1, src, dst)


def _mlp_body(x_ref, a0_ref, a1_ref, degh_ref, w0_ref, b0_ref, w1_ref, b1_ref,
              w2_ref, b2_ref, out_ref):
    deg = jnp.sum(degh_ref[...], axis=1)[:, None]
    inv = 1.0 / jnp.maximum(deg, 1.0)
    agg = jnp.concatenate([a0_ref[...], a1_ref[...]], axis=-1) * inv
    h = jnp.concatenate([x_ref[...], agg], axis=-1)
    h = jnp.maximum(jnp.dot(h, w0_ref[...], preferred_element_type=jnp.float32)
                    + b0_ref[...], 0.0)
    h = jnp.maximum(jnp.dot(h, w1_ref[...], preferred_element_type=jnp.float32)
                    + b1_ref[...], 0.0)
    out_ref[...] = jnp.dot(h, w2_ref[...], preferred_element_type=jnp.float32) \
        + b2_ref[...]


def _mlp(x, a0, a1, degh, W0, b0, W1, b1, W2, b2):
    grid = (_N // _NODE_BLK,)
    return pl.pallas_call(
        _mlp_body,
        grid=grid,
        in_specs=[
            pl.BlockSpec((_NODE_BLK, _IN), lambda i: (i, 0)),
            pl.BlockSpec((_NODE_BLK, _HF), lambda i: (i, 0)),
            pl.BlockSpec((_NODE_BLK, _HF), lambda i: (i, 0)),
            pl.BlockSpec((_NODE_BLK, 16), lambda i: (i, 0)),
            pl.BlockSpec((2 * _IN, _H), lambda i: (0, 0)),
            pl.BlockSpec((1, _H), lambda i: (0, 0)),
            pl.BlockSpec((_H, _H), lambda i: (0, 0)),
            pl.BlockSpec((1, _H), lambda i: (0, 0)),
            pl.BlockSpec((_H, _OUT), lambda i: (0, 0)),
            pl.BlockSpec((1, _OUT), lambda i: (0, 0)),
        ],
        out_specs=pl.BlockSpec((_NODE_BLK, _OUT), lambda i: (i, 0)),
        out_shape=jax.ShapeDtypeStruct((_N, _OUT), jnp.float32),
    )(x, a0, a1, degh, W0, b0.reshape(1, _H), W1, b1.reshape(1, _H),
      W2, b2.reshape(1, _OUT))


def kernel(x, edge_index, edge_attr, W_e, b_e, W0, b0, W1, b1, W2, b2):
    src = edge_index[0]
    dst = edge_index[1]
    x0 = x[:, :_HF]
    x1 = x[:, _HF:]
    e0, e1 = _edge_encoder(edge_attr, W_e, b_e)
    a0, a1, hist = _sc_aggregate(x0, x1, e0, e1, src, dst)
    degh = hist.reshape(16, _NPAD).T
    return _mlp(x, a0[:_N], a1[:_N], degh, W0, b0, W1, b1, W2, b2)


# speedup vs baseline: 3.3660x; 1.4530x over previous
import functools

import jax
import jax.numpy as jnp
from jax import lax
from jax.experimental import pallas as pl
from jax.experimental.pallas import tpu as pltpu
from jax.experimental.pallas import tpu_sc as plsc

_N = 10000
_E = 160000
_IN = 256
_OUT = 256
_ED = 16
_H = 512

_HF = 128
_K = 48
_NCHUNK = 210
_EPT = _K * _NCHUNK
_EPAD = 16 * _EPT
_TRASH = 10016
_NPAD = 10240
_NPT = _NPAD // 16
_DROWS = 40

_EDGE_BLK = 2016
_NODE_BLK = 1000


def _edge_enc_body(ea_ref, we_ref, be_ref, lo_ref, hi_ref):
    e = jnp.dot(ea_ref[...], we_ref[...], preferred_element_type=jnp.float32)
    e = jnp.maximum(e + be_ref[...], 0.0)
    lo_ref[...] = e[:, :_HF]
    hi_ref[...] = e[:, _HF:]


def _edge_encoder(edge_attr_pad, W_e, b_e):
    grid = (_EPAD // _EDGE_BLK,)
    return pl.pallas_call(
        _edge_enc_body,
        grid=grid,
        in_specs=[
            pl.BlockSpec((_EDGE_BLK, _ED), lambda i: (i, 0)),
            pl.BlockSpec((_ED, _IN), lambda i: (0, 0)),
            pl.BlockSpec((1, _IN), lambda i: (0, 0)),
        ],
        out_specs=[
            pl.BlockSpec((_EDGE_BLK, _HF), lambda i: (i, 0)),
            pl.BlockSpec((_EDGE_BLK, _HF), lambda i: (i, 0)),
        ],
        out_shape=[
            jax.ShapeDtypeStruct((_EPAD, _HF), jnp.float32),
            jax.ShapeDtypeStruct((_EPAD, _HF), jnp.float32),
        ],
    )(edge_attr_pad, W_e, b_e.reshape(1, _IN))


def _sc_edges(x_hbm, e_hbm, src_hbm, dst_hbm,
              srcb, dstb, dsc, xg, eb, degl, acc,
              si, se, sg, ss, s, do_deg):
    ebase = s * _EPT
    lanes = lax.iota(jnp.int32, 16)
    onev = jnp.ones((16,), jnp.float32)

    def issue_idx_e(jv, p2):
        b = ebase + jv * _K
        pltpu.async_copy(src_hbm.at[pl.ds(b, _K)], srcb.at[p2], si[p2])
        pltpu.async_copy(dst_hbm.at[pl.ds(b, _K)], dstb.at[p2], si[p2])
        pltpu.async_copy(e_hbm.at[pl.ds(b, _K)], eb.at[p2], se[p2])

    def wait_idx(jv, p2):
        b = ebase + jv * _K
        pltpu.make_async_copy(src_hbm.at[pl.ds(b, _K)], srcb.at[p2],
                              si[p2]).wait()
        pltpu.make_async_copy(dst_hbm.at[pl.ds(b, _K)], dstb.at[p2],
                              si[p2]).wait()

    def issue_gather(p2, p3):
        pltpu.async_copy(x_hbm.at[srcb.at[p2]], xg.at[p3], sg[p3])

    def wait_gather(p2, p3):
        pltpu.make_async_copy(x_hbm.at[srcb.at[p2]], xg.at[p3],
                              sg[p3]).wait()

    def wait_scatter(p3):
        pltpu.make_async_copy(xg.at[p3], acc.at[dsc.at[p3]], ss[p3]).wait()

    def emit(jv, i6, issue_next, wait_sc):
        p2, p3 = i6 % 2, i6 % 3
        n2, n3 = (i6 + 1) % 2, (i6 + 1) % 3
        b = ebase + jv * _K
        if issue_next:
            issue_idx_e(jv + 1, n2)
        if do_deg:
            for v in range(_K // 16):
                dv = dstb[p2, pl.ds(v * 16, 16)]
                for ln in range(16):
                    plsc.addupdate_scatter(degl, [dv], onev,
                                           mask=lanes == ln)
        for v in range(_K // 16):
            dsc[p3, pl.ds(v * 16, 16)] = dstb[p2, pl.ds(v * 16, 16)]
        pltpu.make_async_copy(e_hbm.at[pl.ds(b, _K)], eb.at[p2],
                              se[p2]).wait()
        wait_gather(p2, p3)
        if issue_next:
            wait_idx(jv + 1, n2)
            if wait_sc:
                wait_scatter(n3)
            issue_gather(n2, n3)

        def rowfn(k, c2):
            for qq in range(_HF // 16):
                sl = pl.ds(qq * 16, 16)
                xg[p3, k, sl] = jnp.maximum(xg[p3, k, sl] + eb[p2, k, sl],
                                            0.0)
            return c2
        lax.fori_loop(0, _K, rowfn, 0, unroll=False)

        pltpu.async_copy(xg.at[p3], acc.at[dsc.at[p3]], ss[p3], add=True)

    issue_idx_e(0, 0)
    wait_idx(0, 0)
    issue_gather(0, 0)
    for j in range(6):
        emit(j, j, True, j >= 2)

    def body6(t, carry):
        j0 = 6 * t
        for i in range(6):
            emit(j0 + i, i, True, True)
        return carry
    lax.fori_loop(1, _NCHUNK // 6 - 1, body6, 0, unroll=False)

    for j in range(_NCHUNK - 6, _NCHUNK):
        emit(j, j % 6, j < _NCHUNK - 1, True)
    for p3 in range(3):
        wait_scatter(p3)


def _zero_init(acc, xg, s):
    zv = jnp.zeros((16,), jnp.float32)

    def zrow(k, carry):
        for q in range(_HF // 16):
            xg[0, k, pl.ds(q * 16, 16)] = zv
        return carry
    lax.fori_loop(0, _DROWS, zrow, 0, unroll=False)

    nbase = s * _NPT
    def zinit(i, carry):
        pltpu.sync_copy(xg.at[0, pl.ds(0, _DROWS)],
                        acc.at[pl.ds(nbase + i * _DROWS, _DROWS)])
        return carry
    lax.fori_loop(0, _NPT // _DROWS, zinit, 0, unroll=False)


def _dump_acc(acc, out_hbm, xg, s):
    nbase = s * _NPT
    def dump(i, carry):
        r = nbase + i * _DROWS
        pltpu.sync_copy(acc.at[pl.ds(r, _DROWS)], xg.at[0, pl.ds(0, _DROWS)])
        pltpu.sync_copy(xg.at[0, pl.ds(0, _DROWS)],
                        out_hbm.at[pl.ds(r, _DROWS)])
        return carry
    lax.fori_loop(0, _NPT // _DROWS, dump, 0, unroll=False)


def _sc_body(x0, x1, e0, e1, src_hbm, dst_hbm, out0, out1, outh,
             srcb, dstb, dsc, xg, eb, degl,
             si0, si1, se0, se1, sg0, sg1, sg2, ss0, ss1, ss2, acc):
    c = lax.axis_index("c")
    s = lax.axis_index("s")
    si = (si0, si1)
    se = (se0, se1)
    sg = (sg0, sg1, sg2)
    ss = (ss0, ss1, ss2)

    _zero_init(acc, xg, s)

    @pl.when(c == 0)
    def _():
        zv = jnp.zeros((16,), jnp.float32)
        def zdeg(i, carry):
            degl[pl.ds(i * 16, 16)] = zv
            return carry
        lax.fori_loop(0, _NPAD // 16, zdeg, 0, unroll=False)

    plsc.subcore_barrier()

    @pl.when(c == 0)
    def _():
        _sc_edges(x0, e0, src_hbm, dst_hbm, srcb, dstb, dsc, xg, eb,
                  degl, acc, si, se, sg, ss, s, True)

    @pl.when(c == 1)
    def _():
        _sc_edges(x1, e1, src_hbm, dst_hbm, srcb, dstb, dsc, xg, eb,
                  degl, acc, si, se, sg, ss, s, False)

    plsc.subcore_barrier()

    @pl.when(c == 0)
    def _():
        _dump_acc(acc, out0, xg, s)
        pltpu.sync_copy(degl, outh.at[pl.ds(s * _NPAD, _NPAD)])

    @pl.when(c == 1)
    def _():
        _dump_acc(acc, out1, xg, s)


def _sc_aggregate(x0, x1, e0, e1, src, dst):
    mesh = plsc.VectorSubcoreMesh(core_axis_name="c", subcore_axis_name="s")
    fn = functools.partial(
        pl.kernel,
        out_type=[
            jax.ShapeDtypeStruct((_NPAD, _HF), jnp.float32),
            jax.ShapeDtypeStruct((_NPAD, _HF), jnp.float32),
            jax.ShapeDtypeStruct((16 * _NPAD,), jnp.float32),
        ],
        mesh=mesh,
        compiler_params=pltpu.CompilerParams(needs_layout_passes=False),
        scratch_types=[
            pltpu.VMEM((2, _K), jnp.int32),
            pltpu.VMEM((2, _K), jnp.int32),
            pltpu.VMEM((3, _K), jnp.int32),
            pltpu.VMEM((3, _K, _HF), jnp.float32),
            pltpu.VMEM((2, _K, _HF), jnp.float32),
            pltpu.VMEM((_NPAD,), jnp.float32),
            pltpu.SemaphoreType.DMA,
            pltpu.SemaphoreType.DMA,
            pltpu.SemaphoreType.DMA,
            pltpu.SemaphoreType.DMA,
            pltpu.SemaphoreType.DMA,
            pltpu.SemaphoreType.DMA,
            pltpu.SemaphoreType.DMA,
            pltpu.SemaphoreType.DMA,
            pltpu.SemaphoreType.DMA,
            pltpu.SemaphoreType.DMA,
            pltpu.VMEM_SHARED((_NPAD, _HF), jnp.float32),
        ],
    )(_sc_body)
    return fn(x0, x1, e0, e1, src, dst)


def _mlp_body(x_ref, a0_ref, a1_ref, degh_ref, w0_ref, b0_ref, w1_ref, b1_ref,
              w2_ref, b2_ref, out_ref):
    deg = jnp.sum(degh_ref[...], axis=1)[:, None]
    inv = 1.0 / jnp.maximum(deg, 1.0)
    agg = jnp.concatenate([a0_ref[...], a1_ref[...]], axis=-1) * inv
    h = jnp.concatenate([x_ref[...], agg], axis=-1)
    h = jnp.maximum(jnp.dot(h, w0_ref[...], preferred_element_type=jnp.float32)
                    + b0_ref[...], 0.0)
    h = jnp.maximum(jnp.dot(h, w1_ref[...], preferred_element_type=jnp.float32)
                    + b1_ref[...], 0.0)
    out_ref[...] = jnp.dot(h, w2_ref[...], preferred_element_type=jnp.float32) \
        + b2_ref[...]


def _mlp(x, a0, a1, degh, W0, b0, W1, b1, W2, b2):
    grid = (_N // _NODE_BLK,)
    return pl.pallas_call(
        _mlp_body,
        grid=grid,
        in_specs=[
            pl.BlockSpec((_NODE_BLK, _IN), lambda i: (i, 0)),
            pl.BlockSpec((_NODE_BLK, _HF), lambda i: (i, 0)),
            pl.BlockSpec((_NODE_BLK, _HF), lambda i: (i, 0)),
            pl.BlockSpec((_NODE_BLK, 16), lambda i: (i, 0)),
            pl.BlockSpec((2 * _IN, _H), lambda i: (0, 0)),
            pl.BlockSpec((1, _H), lambda i: (0, 0)),
            pl.BlockSpec((_H, _H), lambda i: (0, 0)),
            pl.BlockSpec((1, _H), lambda i: (0, 0)),
            pl.BlockSpec((_H, _OUT), lambda i: (0, 0)),
            pl.BlockSpec((1, _OUT), lambda i: (0, 0)),
        ],
        out_specs=pl.BlockSpec((_NODE_BLK, _OUT), lambda i: (i, 0)),
        out_shape=jax.ShapeDtypeStruct((_N, _OUT), jnp.float32),
    )(x, a0, a1, degh, W0, b0.reshape(1, _H), W1, b1.reshape(1, _H),
      W2, b2.reshape(1, _OUT))


def kernel(x, edge_index, edge_attr, W_e, b_e, W0, b0, W1, b1, W2, b2):
    npad = _EPAD - _E
    src = jnp.concatenate([edge_index[0],
                           jnp.zeros((npad,), jnp.int32)])
    dst = jnp.concatenate([edge_index[1],
                           jnp.full((npad,), _TRASH, jnp.int32)])
    ea_pad = jnp.concatenate(
        [edge_attr, jnp.zeros((npad, _ED), jnp.float32)])
    x0 = x[:, :_HF]
    x1 = x[:, _HF:]
    e0, e1 = _edge_encoder(ea_pad, W_e, b_e)
    a0, a1, hist = _sc_aggregate(x0, x1, e0, e1, src, dst)
    degh = hist.reshape(16, _NPAD).T
    return _mlp(x, a0[:_N], a1[:_N], degh, W0, b0, W1, b1, W2, b2)
